# baseline (speedup 1.0000x reference)
import functools

import jax
import jax.numpy as jnp
from jax import lax
from jax.experimental import pallas as pl
from jax.experimental.pallas import tpu as pltpu
from jax.experimental.pallas import tpu_sc as plsc

NC = 2
NS = 16
NW = NC * NS
L = 16

H = 128
HJ = H // L
CHUNK = 256
IDXL = 128
NTOK = 204800
NCHUNK = NTOK // NW // CHUNK


def _rsqrt_vec(v):
    i = plsc.bitcast(v, jnp.int32)
    i = jnp.int32(0x5F3759DF) - lax.shift_right_logical(i, 1)
    y = plsc.bitcast(i, jnp.float32)
    for _ in range(2):
        y = y * (jnp.float32(1.5) - jnp.float32(0.5) * v * y * y)
    return y


def _sc_body(word_hbm, pos_hbm, tt_hbm, at_hbm, gamma_hbm, beta_hbm,
             ids3_hbm, out_hbm,
             ib0, ib1, wbuf0, wbuf1,
             pos_v, ttb, atb, combo_v, gm_v, bt_v,
             gsem0, gsem1, wsem, isem):
    wid = lax.axis_index("s") * NC + lax.axis_index("c")
    base = wid * (NTOK // NW)

    pltpu.sync_copy(pos_hbm.at[pl.ds(0, 200)], pos_v)
    pltpu.sync_copy(tt_hbm, ttb)
    pltpu.sync_copy(at_hbm, atb)
    pltpu.sync_copy(gamma_hbm, gm_v)
    pltpu.sync_copy(beta_hbm, bt_v)

    for t in range(2):
        for a in range(8):
            for j in range(HJ):
                combo_v[t * 8 + a, j * L:(j + 1) * L] = (
                    ttb[t, j * L:(j + 1) * L] + atb[a, j * L:(j + 1) * L])

    gms = [gm_v[j * L:(j + 1) * L] for j in range(HJ)]
    bts = [bt_v[j * L:(j + 1) * L] for j in range(HJ)]

    bufs = ((ib0, wbuf0, gsem0), (ib1, wbuf1, gsem1))

    def issue_gathers(ib, wb, gs):
        pltpu.async_copy(word_hbm.at[ib.at[0, pl.ds(0, IDXL)]],
                         wb.at[pl.ds(0, IDXL)], gs)
        pltpu.async_copy(word_hbm.at[ib.at[0, pl.ds(IDXL, IDXL)]],
                         wb.at[pl.ds(IDXL, IDXL)], gs)

    def wait_gathers(ib, wb, gs):
        pltpu.make_async_copy(word_hbm.at[ib.at[0, pl.ds(0, IDXL)]],
                              wb.at[pl.ds(0, IDXL)], gs).wait()
        pltpu.make_async_copy(word_hbm.at[ib.at[0, pl.ds(IDXL, IDXL)]],
                              wb.at[pl.ds(IDXL, IDXL)], gs).wait()

    def compute_chunk(wb, ib, off):
        @plsc.parallel_loop(0, CHUNK // L, 1)
        def group_body(g):
            t0 = g * L
            ttg = ib[1, pl.ds(t0, L)]
            atg = ib[2, pl.ds(t0, L)]
            civ = ttg * 8 + atg
            s0 = lax.rem(off + t0, jnp.int32(200))
            for k in range(L):
                t = t0 + k
                cid = civ[k]
                sk = s0 + k
                s = lax.select(sk >= 200, sk - 200, sk)
                acc = None
                accq = None
                for j in range(HJ):
                    sl = pl.ds(j * L, L)
                    x = wb[t, sl] + combo_v[cid, sl] + pos_v[s, sl]
                    wb[t, sl] = x
                    acc = x if acc is None else acc + x
                    accq = x * x if accq is None else accq + x * x
                tv = jnp.full((L,), jnp.sum(acc), jnp.float32)
                qv = jnp.full((L,), jnp.sum(accq), jnp.float32)
                mean = tv * jnp.float32(1.0 / H)
                var = qv * jnp.float32(1.0 / H) - mean * mean
                av = _rsqrt_vec(var + jnp.float32(1e-12))
                bv = -mean * av
                for j in range(HJ):
                    sl = pl.ds(j * L, L)
                    y = (wb[t, sl] * av + bv) * gms[j] + bts[j]
                    wb[t, sl] = y

    pltpu.sync_copy(ids3_hbm.at[:, pl.ds(base, CHUNK)], ib0)
    pltpu.async_copy(ids3_hbm.at[:, pl.ds(base + CHUNK, CHUNK)], ib1, isem)
    issue_gathers(ib0, wbuf0, gsem0)

    def outer(i, carry):
        for b in range(2):
            ib, wb, gs = bufs[b]
            oib, owb, ogs = bufs[1 - b]
            c = i * 2 + b
            off = base + c * CHUNK

            @pl.when(jnp.logical_and(c >= 1, c < NCHUNK - 1))
            def _():
                pltpu.make_async_copy(
                    owb, out_hbm.at[pl.ds(base, CHUNK)], wsem).wait()

            wait_gathers(ib, wb, gs)

            @pl.when(c < NCHUNK - 1)
            def _():
                pltpu.make_async_copy(
                    ids3_hbm.at[:, pl.ds(base, CHUNK)], oib, isem).wait()
                issue_gathers(oib, owb, ogs)

            compute_chunk(wb, ib, off)
            pltpu.async_copy(wb, out_hbm.at[pl.ds(off, CHUNK)], wsem)

            @pl.when(c < NCHUNK - 2)
            def _():
                pltpu.async_copy(
                    ids3_hbm.at[:, pl.ds(off + 2 * CHUNK, CHUNK)], ib, isem)
        return carry

    lax.fori_loop(0, NCHUNK // 2, outer, 0)

    last_off = base + (NCHUNK - 1) * CHUNK
    wait_gathers(ib0, wbuf0, gsem0)
    compute_chunk(wbuf0, ib0, last_off)
    pltpu.async_copy(wbuf0, out_hbm.at[pl.ds(last_off, CHUNK)], wsem)

    pltpu.make_async_copy(wbuf0, out_hbm.at[pl.ds(base, CHUNK)], wsem).wait()
    pltpu.make_async_copy(wbuf1, out_hbm.at[pl.ds(base, CHUNK)], wsem).wait()


def kernel(word_emb, position_emb, token_type_emb, atom_type_emb,
           ln_gamma, ln_beta, input_ids, token_type_ids, atom_type_ids):
    B, S = input_ids.shape
    N = B * S
    ids3 = jnp.stack([input_ids.reshape(N), token_type_ids.reshape(N),
                      atom_type_ids.reshape(N)])

    mesh = plsc.VectorSubcoreMesh(core_axis_name="c", subcore_axis_name="s")
    k = pl.kernel(
        _sc_body,
        mesh=mesh,
        compiler_params=pltpu.CompilerParams(needs_layout_passes=False),
        out_type=jax.ShapeDtypeStruct((N, H), jnp.float32),
        scratch_types=[
            pltpu.VMEM((3, CHUNK), jnp.int32),
            pltpu.VMEM((3, CHUNK), jnp.int32),
            pltpu.VMEM((CHUNK, H), jnp.float32),
            pltpu.VMEM((CHUNK, H), jnp.float32),
            pltpu.VMEM((200, H), jnp.float32),
            pltpu.VMEM((2, H), jnp.float32),
            pltpu.VMEM((8, H), jnp.float32),
            pltpu.VMEM((16, H), jnp.float32),
            pltpu.VMEM((H,), jnp.float32),
            pltpu.VMEM((H,), jnp.float32),
            pltpu.SemaphoreType.DMA,
            pltpu.SemaphoreType.DMA,
            pltpu.SemaphoreType.DMA,
            pltpu.SemaphoreType.DMA,
        ],
    )
    out = k(word_emb, position_emb, token_type_emb, atom_type_emb,
            ln_gamma, ln_beta, ids3)
    return out.reshape(B, S, H)


# speedup vs baseline: 1.5345x; 1.5345x over previous
import functools

import jax
import jax.numpy as jnp
from jax import lax
from jax.experimental import pallas as pl
from jax.experimental.pallas import tpu as pltpu
from jax.experimental.pallas import tpu_sc as plsc

NC = 2
NS = 16
NW = NC * NS
L = 16

H = 128
HJ = H // L
CHUNK = 256
IDXL = 128
NTOK = 204800
NCHUNK = NTOK // NW // CHUNK


def _rsqrt_vec(v):
    i = plsc.bitcast(v, jnp.int32)
    i = jnp.int32(0x5F3759DF) - lax.shift_right_logical(i, 1)
    y = plsc.bitcast(i, jnp.float32)
    for _ in range(2):
        y = y * (jnp.float32(1.5) - jnp.float32(0.5) * v * y * y)
    return y


def _sc_body(word_hbm, pos_hbm, tt_hbm, at_hbm, gamma_hbm, beta_hbm,
             ids3_hbm, out_hbm,
             ib0, ib1, wbuf0, wbuf1,
             pos_v, ttb, atb, combo_v, gm_v, bt_v,
             gsem0, gsem1, wsem, isem):
    wid = lax.axis_index("s") * NC + lax.axis_index("c")
    base = wid * (NTOK // NW)

    pltpu.sync_copy(pos_hbm.at[pl.ds(0, 200)], pos_v)
    pltpu.sync_copy(tt_hbm, ttb)
    pltpu.sync_copy(at_hbm, atb)
    pltpu.sync_copy(gamma_hbm, gm_v)
    pltpu.sync_copy(beta_hbm, bt_v)

    for t in range(2):
        for a in range(8):
            for j in range(HJ):
                combo_v[t * 8 + a, j * L:(j + 1) * L] = (
                    ttb[t, j * L:(j + 1) * L] + atb[a, j * L:(j + 1) * L])

    gms = [gm_v[j * L:(j + 1) * L] for j in range(HJ)]
    bts = [bt_v[j * L:(j + 1) * L] for j in range(HJ)]

    bufs = ((ib0, wbuf0, gsem0), (ib1, wbuf1, gsem1))

    def issue_gathers(ib, wb, gs):
        pltpu.async_copy(word_hbm.at[ib.at[0, pl.ds(0, IDXL)]],
                         wb.at[pl.ds(0, IDXL)], gs)
        pltpu.async_copy(word_hbm.at[ib.at[0, pl.ds(IDXL, IDXL)]],
                         wb.at[pl.ds(IDXL, IDXL)], gs)

    def wait_gathers(ib, wb, gs):
        pltpu.make_async_copy(word_hbm.at[ib.at[0, pl.ds(0, IDXL)]],
                              wb.at[pl.ds(0, IDXL)], gs).wait()
        pltpu.make_async_copy(word_hbm.at[ib.at[0, pl.ds(IDXL, IDXL)]],
                              wb.at[pl.ds(IDXL, IDXL)], gs).wait()

    def compute_chunk(wb, ib, off):
        @plsc.parallel_loop(0, CHUNK // L, 1)
        def group_body(g):
            t0 = g * L
            ttg = ib[1, pl.ds(t0, L)]
            atg = ib[2, pl.ds(t0, L)]
            civ = ttg * 8 + atg
            s0 = lax.rem(off + t0, jnp.int32(200))
            for k in range(L):
                t = t0 + k
                cid = civ[k]
                sk = s0 + k
                s = lax.select(sk >= 200, sk - 200, sk)
                xs = []
                for j in range(HJ):
                    sl = pl.ds(j * L, L)
                    x = wb[t, sl] + combo_v[cid, sl] + pos_v[s, sl]
                    xs.append(x)
                ss = xs
                qs = [x * x for x in xs]
                while len(ss) > 1:
                    ss = [ss[i] + ss[i + 1] for i in range(0, len(ss), 2)]
                    qs = [qs[i] + qs[i + 1] for i in range(0, len(qs), 2)]
                tv = jnp.full((L,), jnp.sum(ss[0]), jnp.float32)
                qv = jnp.full((L,), jnp.sum(qs[0]), jnp.float32)
                mean = tv * jnp.float32(1.0 / H)
                var = qv * jnp.float32(1.0 / H) - mean * mean
                av = _rsqrt_vec(var + jnp.float32(1e-12))
                bv = -mean * av
                for j in range(HJ):
                    y = (xs[j] * av + bv) * gms[j] + bts[j]
                    wb[t, pl.ds(j * L, L)] = y

    pltpu.sync_copy(ids3_hbm.at[:, pl.ds(base, CHUNK)], ib0)
    pltpu.async_copy(ids3_hbm.at[:, pl.ds(base + CHUNK, CHUNK)], ib1, isem)
    issue_gathers(ib0, wbuf0, gsem0)

    def outer(i, carry):
        for b in range(2):
            ib, wb, gs = bufs[b]
            oib, owb, ogs = bufs[1 - b]
            c = i * 2 + b
            off = base + c * CHUNK

            @pl.when(jnp.logical_and(c >= 1, c < NCHUNK - 1))
            def _():
                pltpu.make_async_copy(
                    owb, out_hbm.at[pl.ds(base, CHUNK)], wsem).wait()

            wait_gathers(ib, wb, gs)

            @pl.when(c < NCHUNK - 1)
            def _():
                pltpu.make_async_copy(
                    ids3_hbm.at[:, pl.ds(base, CHUNK)], oib, isem).wait()
                issue_gathers(oib, owb, ogs)

            compute_chunk(wb, ib, off)
            pltpu.async_copy(wb, out_hbm.at[pl.ds(off, CHUNK)], wsem)

            @pl.when(c < NCHUNK - 2)
            def _():
                pltpu.async_copy(
                    ids3_hbm.at[:, pl.ds(off + 2 * CHUNK, CHUNK)], ib, isem)
        return carry

    lax.fori_loop(0, NCHUNK // 2, outer, 0)

    last_off = base + (NCHUNK - 1) * CHUNK
    wait_gathers(ib0, wbuf0, gsem0)
    compute_chunk(wbuf0, ib0, last_off)
    pltpu.async_copy(wbuf0, out_hbm.at[pl.ds(last_off, CHUNK)], wsem)

    pltpu.make_async_copy(wbuf0, out_hbm.at[pl.ds(base, CHUNK)], wsem).wait()
    pltpu.make_async_copy(wbuf1, out_hbm.at[pl.ds(base, CHUNK)], wsem).wait()


def kernel(word_emb, position_emb, token_type_emb, atom_type_emb,
           ln_gamma, ln_beta, input_ids, token_type_ids, atom_type_ids):
    B, S = input_ids.shape
    N = B * S
    ids3 = jnp.stack([input_ids.reshape(N), token_type_ids.reshape(N),
                      atom_type_ids.reshape(N)])

    mesh = plsc.VectorSubcoreMesh(core_axis_name="c", subcore_axis_name="s")
    k = pl.kernel(
        _sc_body,
        mesh=mesh,
        compiler_params=pltpu.CompilerParams(needs_layout_passes=False),
        out_type=jax.ShapeDtypeStruct((N, H), jnp.float32),
        scratch_types=[
            pltpu.VMEM((3, CHUNK), jnp.int32),
            pltpu.VMEM((3, CHUNK), jnp.int32),
            pltpu.VMEM((CHUNK, H), jnp.float32),
            pltpu.VMEM((CHUNK, H), jnp.float32),
            pltpu.VMEM((200, H), jnp.float32),
            pltpu.VMEM((2, H), jnp.float32),
            pltpu.VMEM((8, H), jnp.float32),
            pltpu.VMEM((16, H), jnp.float32),
            pltpu.VMEM((H,), jnp.float32),
            pltpu.VMEM((H,), jnp.float32),
            pltpu.SemaphoreType.DMA,
            pltpu.SemaphoreType.DMA,
            pltpu.SemaphoreType.DMA,
            pltpu.SemaphoreType.DMA,
        ],
    )
    out = k(word_emb, position_emb, token_type_emb, atom_type_emb,
            ln_gamma, ln_beta, ids3)
    return out.reshape(B, S, H)


# speedup vs baseline: 1.5757x; 1.0269x over previous
import functools

import jax
import jax.numpy as jnp
from jax import lax
from jax.experimental import pallas as pl
from jax.experimental.pallas import tpu as pltpu
from jax.experimental.pallas import tpu_sc as plsc

NC = 2
NS = 16
NW = NC * NS
L = 16

H = 128
HJ = H // L
CHUNK = 256
IDXL = 128
NTOK = 204800
NCHUNK = NTOK // NW // CHUNK


def _rsqrt_vec(v):
    i = plsc.bitcast(v, jnp.int32)
    i = jnp.int32(0x5F3759DF) - lax.shift_right_logical(i, 1)
    y = plsc.bitcast(i, jnp.float32)
    for _ in range(2):
        y = y * (jnp.float32(1.5) - jnp.float32(0.5) * v * y * y)
    return y


def _sc_body(word_hbm, pos_hbm, tt_hbm, at_hbm, gamma_hbm, beta_hbm,
             ids3_hbm, out_hbm,
             ib0, ib1, wbuf0, wbuf1,
             pos_v, ttb, atb, combo_v, gm_v, bt_v,
             gsem0, gsem1, wsem, isem):
    wid = lax.axis_index("s") * NC + lax.axis_index("c")
    base = wid * (NTOK // NW)

    pltpu.sync_copy(pos_hbm.at[pl.ds(0, 200)], pos_v)
    pltpu.sync_copy(tt_hbm, ttb)
    pltpu.sync_copy(at_hbm, atb)
    pltpu.sync_copy(gamma_hbm, gm_v)
    pltpu.sync_copy(beta_hbm, bt_v)

    for t in range(2):
        for a in range(8):
            for j in range(HJ):
                combo_v[t * 8 + a, j * L:(j + 1) * L] = (
                    ttb[t, j * L:(j + 1) * L] + atb[a, j * L:(j + 1) * L])

    gms = [gm_v[j * L:(j + 1) * L] for j in range(HJ)]
    bts = [bt_v[j * L:(j + 1) * L] for j in range(HJ)]

    bufs = ((ib0, wbuf0, gsem0), (ib1, wbuf1, gsem1))

    def issue_gathers(ib, wb, gs):
        pltpu.async_copy(word_hbm.at[ib.at[0, pl.ds(0, IDXL)]],
                         wb.at[pl.ds(0, IDXL)], gs)
        pltpu.async_copy(word_hbm.at[ib.at[0, pl.ds(IDXL, IDXL)]],
                         wb.at[pl.ds(IDXL, IDXL)], gs)

    def wait_gathers(ib, wb, gs):
        pltpu.make_async_copy(word_hbm.at[ib.at[0, pl.ds(0, IDXL)]],
                              wb.at[pl.ds(0, IDXL)], gs).wait()
        pltpu.make_async_copy(word_hbm.at[ib.at[0, pl.ds(IDXL, IDXL)]],
                              wb.at[pl.ds(IDXL, IDXL)], gs).wait()

    def compute_chunk(wb, ib, off):
        @plsc.parallel_loop(0, CHUNK // L, 1)
        def group_body(g):
            t0 = g * L
            ttg = ib[1, pl.ds(t0, L)]
            atg = ib[2, pl.ds(t0, L)]
            civ = ttg * 8 + atg
            s0 = lax.rem(off + t0, jnp.int32(200))
            for k in range(L):
                t = t0 + k
                cid = civ[k]
                sk = s0 + k
                s = lax.select(sk >= 200, sk - 200, sk)
                xs = []
                acc = None
                accq = None
                for j in range(HJ):
                    sl = pl.ds(j * L, L)
                    x = wb[t, sl] + combo_v[cid, sl] + pos_v[s, sl]
                    xs.append(x)
                    acc = x if acc is None else acc + x
                    accq = x * x if accq is None else accq + x * x
                tv = jnp.full((L,), jnp.sum(acc), jnp.float32)
                qv = jnp.full((L,), jnp.sum(accq), jnp.float32)
                mean = tv * jnp.float32(1.0 / H)
                var = qv * jnp.float32(1.0 / H) - mean * mean
                av = _rsqrt_vec(var + jnp.float32(1e-12))
                bv = -mean * av
                for j in range(HJ):
                    y = (xs[j] * av + bv) * gms[j] + bts[j]
                    wb[t, pl.ds(j * L, L)] = y

    pltpu.sync_copy(ids3_hbm.at[:, pl.ds(base, CHUNK)], ib0)
    pltpu.async_copy(ids3_hbm.at[:, pl.ds(base + CHUNK, CHUNK)], ib1, isem)
    issue_gathers(ib0, wbuf0, gsem0)

    def outer(i, carry):
        for b in range(2):
            ib, wb, gs = bufs[b]
            oib, owb, ogs = bufs[1 - b]
            c = i * 2 + b
            off = base + c * CHUNK

            @pl.when(jnp.logical_and(c >= 1, c < NCHUNK - 1))
            def _():
                pltpu.make_async_copy(
                    owb, out_hbm.at[pl.ds(base, CHUNK)], wsem).wait()

            wait_gathers(ib, wb, gs)

            @pl.when(c < NCHUNK - 1)
            def _():
                pltpu.make_async_copy(
                    ids3_hbm.at[:, pl.ds(base, CHUNK)], oib, isem).wait()
                issue_gathers(oib, owb, ogs)

            compute_chunk(wb, ib, off)
            pltpu.async_copy(wb, out_hbm.at[pl.ds(off, CHUNK)], wsem)

            @pl.when(c < NCHUNK - 2)
            def _():
                pltpu.async_copy(
                    ids3_hbm.at[:, pl.ds(off + 2 * CHUNK, CHUNK)], ib, isem)
        return carry

    lax.fori_loop(0, NCHUNK // 2, outer, 0)

    last_off = base + (NCHUNK - 1) * CHUNK
    wait_gathers(ib0, wbuf0, gsem0)
    compute_chunk(wbuf0, ib0, last_off)
    pltpu.async_copy(wbuf0, out_hbm.at[pl.ds(last_off, CHUNK)], wsem)

    pltpu.make_async_copy(wbuf0, out_hbm.at[pl.ds(base, CHUNK)], wsem).wait()
    pltpu.make_async_copy(wbuf1, out_hbm.at[pl.ds(base, CHUNK)], wsem).wait()


def kernel(word_emb, position_emb, token_type_emb, atom_type_emb,
           ln_gamma, ln_beta, input_ids, token_type_ids, atom_type_ids):
    B, S = input_ids.shape
    N = B * S
    ids3 = jnp.stack([input_ids.reshape(N), token_type_ids.reshape(N),
                      atom_type_ids.reshape(N)])

    mesh = plsc.VectorSubcoreMesh(core_axis_name="c", subcore_axis_name="s")
    k = pl.kernel(
        _sc_body,
        mesh=mesh,
        compiler_params=pltpu.CompilerParams(needs_layout_passes=False),
        out_type=jax.ShapeDtypeStruct((N, H), jnp.float32),
        scratch_types=[
            pltpu.VMEM((3, CHUNK), jnp.int32),
            pltpu.VMEM((3, CHUNK), jnp.int32),
            pltpu.VMEM((CHUNK, H), jnp.float32),
            pltpu.VMEM((CHUNK, H), jnp.float32),
            pltpu.VMEM((200, H), jnp.float32),
            pltpu.VMEM((2, H), jnp.float32),
            pltpu.VMEM((8, H), jnp.float32),
            pltpu.VMEM((16, H), jnp.float32),
            pltpu.VMEM((H,), jnp.float32),
            pltpu.VMEM((H,), jnp.float32),
            pltpu.SemaphoreType.DMA,
            pltpu.SemaphoreType.DMA,
            pltpu.SemaphoreType.DMA,
            pltpu.SemaphoreType.DMA,
        ],
    )
    out = k(word_emb, position_emb, token_type_emb, atom_type_emb,
            ln_gamma, ln_beta, ids3)
    return out.reshape(B, S, H)
